# Initial kernel scaffold; baseline (speedup 1.0000x reference)
import jax
import jax.numpy as jnp
from jax.experimental import pallas as pl


def kernel(traj_loc, mat2, vec, traj_len, l_max, emb_su, emb_sl, emb_tu, emb_tl):
    raise NotImplementedError("write your pallas kernel here")


# speedup vs baseline: 15.5825x; 15.5825x over previous
import jax
import jax.numpy as jnp
from jax.experimental import pallas as pl
from jax.experimental.pallas import tpu as pltpu

_SU, _SL, _TU, _TL = 1000.0, 0.0, 500.0, 0.0
_TM = 50


def _embed_kernel(loc_ref, len_ref, lmax_ref,
                  mat2_ref, su_ref, sl_ref, tu_ref, tl_ref,
                  out_ref, ds_ref):
    b = pl.program_id(0)
    j = pl.program_id(1)
    tm, l = ds_ref.shape
    m0 = j * tm

    for t in range(tm):
        idx = loc_ref[b, m0 + t] - 1
        ds_ref[pl.ds(t, 1), :] = mat2_ref[pl.ds(idx, 1), :]

    tlen = len_ref[b]
    lmax = lmax_ref[0]
    v2 = (jax.lax.broadcasted_iota(jnp.int32, (tm, 1), 0) + m0) < tlen
    col_ok = jax.lax.broadcasted_iota(jnp.int32, (tm, l), 1) < lmax
    ds = jnp.where(v2 & col_ok, ds_ref[...], 0.0)

    esl = jnp.where(v2, sl_ref[1:2, :], sl_ref[0:1, :])
    esu = jnp.where(v2, su_ref[1:2, :], su_ref[0:1, :])
    etl = jnp.where(v2, tl_ref[1:2, :], tl_ref[0:1, :])
    etu = jnp.where(v2, tu_ref[1:2, :], tu_ref[0:1, :])
    a_v = (esl * _SU - esu * _SL) * (1.0 / (_SU - _SL)) + \
          (etl * _TU - etu * _TL) * (1.0 / (_TU - _TL))
    b_v = (esu - esl) * (1.0 / (_SU - _SL)) + \
          (etu - etl) * (1.0 / (_TU - _TL))

    out_ref[0] = a_v[:, None, :] + b_v[:, None, :] * ds[:, :, None]


def kernel(traj_loc, mat2, vec, traj_len, l_max, emb_su, emb_sl, emb_tu, emb_tl):
    del vec
    b_sz, m_sz = traj_loc.shape
    n_loc, l_sz = mat2.shape
    e_sz = emb_su.shape[1]
    tm = _TM if m_sz % _TM == 0 else m_sz
    grid = (b_sz, m_sz // tm)

    lmax_arr = jnp.asarray(l_max, jnp.int32).reshape(1)
    full = lambda bb, jj, *refs: (0, 0)

    out = pl.pallas_call(
        _embed_kernel,
        grid_spec=pltpu.PrefetchScalarGridSpec(
            num_scalar_prefetch=3,
            grid=grid,
            in_specs=[
                pl.BlockSpec((n_loc, l_sz), full),
                pl.BlockSpec((2, e_sz), full),
                pl.BlockSpec((2, e_sz), full),
                pl.BlockSpec((2, e_sz), full),
                pl.BlockSpec((2, e_sz), full),
            ],
            out_specs=pl.BlockSpec((1, tm, l_sz, e_sz),
                                   lambda bb, jj, *refs: (bb, jj, 0, 0)),
            scratch_shapes=[pltpu.VMEM((tm, l_sz), jnp.float32)],
        ),
        out_shape=jax.ShapeDtypeStruct((b_sz, m_sz, l_sz, e_sz), jnp.float32),
    )(traj_loc.astype(jnp.int32), traj_len.astype(jnp.int32), lmax_arr,
      mat2, emb_su, emb_sl, emb_tu, emb_tl)
    return out
